# baseline (device time: 162843 ns/iter reference)
import jax
import jax.numpy as jnp
from jax import lax
from jax.experimental import pallas as pl
from jax.experimental.pallas import tpu as pltpu

N_DEV = 16


def kernel(x, W1, W2):
    m, k = x.shape
    _, h = W1.shape
    _, n = W2.shape
    chunk = m // N_DEV

    def body(x_ref, w1_ref, w2_ref, out_ref,
             part_ref, comm_ref,
             rs_send, rs_recv, ag_send, ag_recv):
        my = lax.axis_index("i")
        left = (my - 1) % N_DEV
        right = (my + 1) % N_DEV

        barrier_sem = pltpu.get_barrier_semaphore()
        for nbr in [left, right]:
            pl.semaphore_signal(
                barrier_sem, inc=1,
                device_id=(nbr,), device_id_type=pl.DeviceIdType.MESH,
            )
        pl.semaphore_wait(barrier_sem, 2)

        hmat = jnp.maximum(
            jnp.dot(x_ref[...], w1_ref[...], preferred_element_type=jnp.float32),
            0.0,
        )
        p = jnp.dot(hmat, w2_ref[...], preferred_element_type=jnp.float32)
        for c in range(N_DEV):
            part_ref[c] = p[c * chunk:(c + 1) * chunk, :]

        for s in range(N_DEV - 1):
            send_idx = (my - s) % N_DEV
            recv_idx = (my - s - 1) % N_DEV
            rdma = pltpu.make_async_remote_copy(
                src_ref=part_ref.at[send_idx],
                dst_ref=comm_ref.at[s],
                send_sem=rs_send.at[s],
                recv_sem=rs_recv.at[s],
                device_id=(right,),
                device_id_type=pl.DeviceIdType.MESH,
            )
            rdma.start()
            rdma.wait()
            part_ref[recv_idx] = part_ref[recv_idx] + comm_ref[s]

        own = (my + 1) % N_DEV
        out_ref[pl.ds(own * chunk, chunk), :] = part_ref[own]
        for t in range(N_DEV - 1):
            send_c = (my + 1 - t) % N_DEV
            rdma = pltpu.make_async_remote_copy(
                src_ref=out_ref.at[pl.ds(send_c * chunk, chunk), :],
                dst_ref=out_ref.at[pl.ds(send_c * chunk, chunk), :],
                send_sem=ag_send.at[t],
                recv_sem=ag_recv.at[t],
                device_id=(right,),
                device_id_type=pl.DeviceIdType.MESH,
            )
            rdma.start()
            rdma.wait()

    return pl.pallas_call(
        body,
        out_shape=jax.ShapeDtypeStruct((m, n), jnp.float32),
        in_specs=[
            pl.BlockSpec(memory_space=pltpu.VMEM),
            pl.BlockSpec(memory_space=pltpu.VMEM),
            pl.BlockSpec(memory_space=pltpu.VMEM),
        ],
        out_specs=pl.BlockSpec(memory_space=pltpu.VMEM),
        scratch_shapes=[
            pltpu.VMEM((N_DEV, chunk, n), jnp.float32),
            pltpu.VMEM((N_DEV - 1, chunk, n), jnp.float32),
            pltpu.SemaphoreType.DMA((N_DEV - 1,)),
            pltpu.SemaphoreType.DMA((N_DEV - 1,)),
            pltpu.SemaphoreType.DMA((N_DEV - 1,)),
            pltpu.SemaphoreType.DMA((N_DEV - 1,)),
        ],
        compiler_params=pltpu.CompilerParams(collective_id=0),
    )(x, W1, W2)


# device time: 161516 ns/iter; 1.0082x vs baseline; 1.0082x over previous
import jax
import jax.numpy as jnp
from jax import lax
from jax.experimental import pallas as pl
from jax.experimental.pallas import tpu as pltpu

N_DEV = 16


def kernel(x, W1, W2):
    m, k = x.shape
    _, h = W1.shape
    _, n = W2.shape
    chunk = m // N_DEV
    half = chunk // 2

    def body(x_ref, w1_ref, w2_ref, out_ref,
             ptop, pbot, ctop, cbot,
             sp_s, sp_r, sm_s, sm_r, ap_s, ap_r, am_s, am_r):
        my = lax.axis_index("i")
        left = (my - 1) % N_DEV
        right = (my + 1) % N_DEV

        barrier_sem = pltpu.get_barrier_semaphore()
        for nbr in [left, right]:
            pl.semaphore_signal(
                barrier_sem, inc=1,
                device_id=(nbr,), device_id_type=pl.DeviceIdType.MESH,
            )
        pl.semaphore_wait(barrier_sem, 2)

        def compute_top(c):
            xs = x_ref[pl.ds(c * chunk, half), :]
            hm = jnp.maximum(
                jnp.dot(xs, w1_ref[...], preferred_element_type=jnp.float32), 0.0)
            ptop[c] = jnp.dot(hm, w2_ref[...], preferred_element_type=jnp.float32)

        def compute_bot(c):
            xs = x_ref[pl.ds(c * chunk + half, half), :]
            hm = jnp.maximum(
                jnp.dot(xs, w1_ref[...], preferred_element_type=jnp.float32), 0.0)
            pbot[c] = jnp.dot(hm, w2_ref[...], preferred_element_type=jnp.float32)

        compute_top(my)
        compute_bot(my)
        compute_top(left)
        compute_bot(right)

        for s in range(N_DEV - 1):
            st = (my - s) % N_DEV
            rt = (my - s - 1) % N_DEV
            sb = (my + s) % N_DEV
            rb = (my + s + 1) % N_DEV
            rp = pltpu.make_async_remote_copy(
                src_ref=ptop.at[st], dst_ref=ctop.at[s],
                send_sem=sp_s.at[s], recv_sem=sp_r.at[s],
                device_id=(right,), device_id_type=pl.DeviceIdType.MESH,
            )
            rm = pltpu.make_async_remote_copy(
                src_ref=pbot.at[sb], dst_ref=cbot.at[s],
                send_sem=sm_s.at[s], recv_sem=sm_r.at[s],
                device_id=(left,), device_id_type=pl.DeviceIdType.MESH,
            )
            rp.start()
            rm.start()
            if s < N_DEV - 2:
                compute_top((my - s - 2) % N_DEV)
                compute_bot((my + s + 2) % N_DEV)
            rp.wait()
            rm.wait()
            ptop[rt] = ptop[rt] + ctop[s]
            pbot[rb] = pbot[rb] + cbot[s]

        ownt = (my + 1) % N_DEV
        ownb = (my - 1) % N_DEV
        out_ref[pl.ds(ownt * chunk, half), :] = ptop[ownt]
        out_ref[pl.ds(ownb * chunk + half, half), :] = pbot[ownb]

        for t in range(N_DEV - 1):
            ct_ = (my + 1 - t) % N_DEV
            cb_ = (my - 1 + t) % N_DEV
            rp = pltpu.make_async_remote_copy(
                src_ref=out_ref.at[pl.ds(ct_ * chunk, half), :],
                dst_ref=out_ref.at[pl.ds(ct_ * chunk, half), :],
                send_sem=ap_s.at[t], recv_sem=ap_r.at[t],
                device_id=(right,), device_id_type=pl.DeviceIdType.MESH,
            )
            rm = pltpu.make_async_remote_copy(
                src_ref=out_ref.at[pl.ds(cb_ * chunk + half, half), :],
                dst_ref=out_ref.at[pl.ds(cb_ * chunk + half, half), :],
                send_sem=am_s.at[t], recv_sem=am_r.at[t],
                device_id=(left,), device_id_type=pl.DeviceIdType.MESH,
            )
            rp.start()
            rm.start()
            rp.wait()
            rm.wait()

    nsteps = N_DEV - 1
    return pl.pallas_call(
        body,
        out_shape=jax.ShapeDtypeStruct((m, n), jnp.float32),
        in_specs=[
            pl.BlockSpec(memory_space=pltpu.VMEM),
            pl.BlockSpec(memory_space=pltpu.VMEM),
            pl.BlockSpec(memory_space=pltpu.VMEM),
        ],
        out_specs=pl.BlockSpec(memory_space=pltpu.VMEM),
        scratch_shapes=[
            pltpu.VMEM((N_DEV, half, n), jnp.float32),
            pltpu.VMEM((N_DEV, half, n), jnp.float32),
            pltpu.VMEM((nsteps, half, n), jnp.float32),
            pltpu.VMEM((nsteps, half, n), jnp.float32),
            pltpu.SemaphoreType.DMA((nsteps,)),
            pltpu.SemaphoreType.DMA((nsteps,)),
            pltpu.SemaphoreType.DMA((nsteps,)),
            pltpu.SemaphoreType.DMA((nsteps,)),
            pltpu.SemaphoreType.DMA((nsteps,)),
            pltpu.SemaphoreType.DMA((nsteps,)),
            pltpu.SemaphoreType.DMA((nsteps,)),
            pltpu.SemaphoreType.DMA((nsteps,)),
        ],
        compiler_params=pltpu.CompilerParams(collective_id=0),
    )(x, W1, W2)


# device time: 120130 ns/iter; 1.3556x vs baseline; 1.3445x over previous
import jax
import jax.numpy as jnp
from jax import lax
from jax.experimental import pallas as pl
from jax.experimental.pallas import tpu as pltpu

N_DEV = 16

_RING = [0, 4, 8, 12, 13, 9, 5, 1, 2, 6, 10, 14, 15, 11, 7, 3]
_POS = [0] * N_DEV
for _r, _p in enumerate(_RING):
    _POS[_p] = _r
_RIGHT = [0] * N_DEV
_LEFT = [0] * N_DEV
for _r, _p in enumerate(_RING):
    _RIGHT[_p] = _RING[(_r + 1) % N_DEV]
    _LEFT[_p] = _RING[(_r - 1) % N_DEV]


def _scalar_map(idx, table):
    import jax.numpy as jnp
    out = jnp.int32(table[0])
    for i in range(1, len(table)):
        out = jnp.where(idx == i, jnp.int32(table[i]), out)
    return out


def kernel(x, W1, W2):
    m, k = x.shape
    _, h = W1.shape
    _, n = W2.shape
    chunk = m // N_DEV
    half = chunk // 2

    def body(x_ref, w1_ref, w2_ref, out_ref,
             ptop, pbot, ctop, cbot,
             sp_s, sp_r, sm_s, sm_r, ap_s, ap_r, am_s, am_r):
        me = lax.axis_index("i")
        my = _scalar_map(me, _POS)
        right = _scalar_map(me, _RIGHT)
        left = _scalar_map(me, _LEFT)

        barrier_sem = pltpu.get_barrier_semaphore()
        for nbr in [left, right]:
            pl.semaphore_signal(
                barrier_sem, inc=1,
                device_id=(nbr,), device_id_type=pl.DeviceIdType.MESH,
            )
        pl.semaphore_wait(barrier_sem, 2)

        def compute_top(c):
            xs = x_ref[pl.ds(c * chunk, half), :]
            hm = jnp.maximum(
                jnp.dot(xs, w1_ref[...], preferred_element_type=jnp.float32), 0.0)
            ptop[c] = jnp.dot(hm, w2_ref[...], preferred_element_type=jnp.float32)

        def compute_bot(c):
            xs = x_ref[pl.ds(c * chunk + half, half), :]
            hm = jnp.maximum(
                jnp.dot(xs, w1_ref[...], preferred_element_type=jnp.float32), 0.0)
            pbot[c] = jnp.dot(hm, w2_ref[...], preferred_element_type=jnp.float32)

        compute_top(my)
        compute_bot(my)
        compute_top((my - 1) % N_DEV)
        compute_bot((my + 1) % N_DEV)

        for s in range(N_DEV - 1):
            st = (my - s) % N_DEV
            rt = (my - s - 1) % N_DEV
            sb = (my + s) % N_DEV
            rb = (my + s + 1) % N_DEV
            rp = pltpu.make_async_remote_copy(
                src_ref=ptop.at[st], dst_ref=ctop.at[s],
                send_sem=sp_s.at[s], recv_sem=sp_r.at[s],
                device_id=(right,), device_id_type=pl.DeviceIdType.MESH,
            )
            rm = pltpu.make_async_remote_copy(
                src_ref=pbot.at[sb], dst_ref=cbot.at[s],
                send_sem=sm_s.at[s], recv_sem=sm_r.at[s],
                device_id=(left,), device_id_type=pl.DeviceIdType.MESH,
            )
            rp.start()
            rm.start()
            if s < N_DEV - 2:
                compute_top((my - s - 2) % N_DEV)
                compute_bot((my + s + 2) % N_DEV)
            rp.wait()
            rm.wait()
            ptop[rt] = ptop[rt] + ctop[s]
            pbot[rb] = pbot[rb] + cbot[s]

        ownt = (my + 1) % N_DEV
        ownb = (my - 1) % N_DEV
        out_ref[pl.ds(ownt * chunk, half), :] = ptop[ownt]
        out_ref[pl.ds(ownb * chunk + half, half), :] = pbot[ownb]

        for t in range(N_DEV - 1):
            ct_ = (my + 1 - t) % N_DEV
            cb_ = (my - 1 + t) % N_DEV
            rp = pltpu.make_async_remote_copy(
                src_ref=out_ref.at[pl.ds(ct_ * chunk, half), :],
                dst_ref=out_ref.at[pl.ds(ct_ * chunk, half), :],
                send_sem=ap_s.at[t], recv_sem=ap_r.at[t],
                device_id=(right,), device_id_type=pl.DeviceIdType.MESH,
            )
            rm = pltpu.make_async_remote_copy(
                src_ref=out_ref.at[pl.ds(cb_ * chunk + half, half), :],
                dst_ref=out_ref.at[pl.ds(cb_ * chunk + half, half), :],
                send_sem=am_s.at[t], recv_sem=am_r.at[t],
                device_id=(left,), device_id_type=pl.DeviceIdType.MESH,
            )
            rp.start()
            rm.start()
            rp.wait()
            rm.wait()

    nsteps = N_DEV - 1
    return pl.pallas_call(
        body,
        out_shape=jax.ShapeDtypeStruct((m, n), jnp.float32),
        in_specs=[
            pl.BlockSpec(memory_space=pltpu.VMEM),
            pl.BlockSpec(memory_space=pltpu.VMEM),
            pl.BlockSpec(memory_space=pltpu.VMEM),
        ],
        out_specs=pl.BlockSpec(memory_space=pltpu.VMEM),
        scratch_shapes=[
            pltpu.VMEM((N_DEV, half, n), jnp.float32),
            pltpu.VMEM((N_DEV, half, n), jnp.float32),
            pltpu.VMEM((nsteps, half, n), jnp.float32),
            pltpu.VMEM((nsteps, half, n), jnp.float32),
            pltpu.SemaphoreType.DMA((nsteps,)),
            pltpu.SemaphoreType.DMA((nsteps,)),
            pltpu.SemaphoreType.DMA((nsteps,)),
            pltpu.SemaphoreType.DMA((nsteps,)),
            pltpu.SemaphoreType.DMA((nsteps,)),
            pltpu.SemaphoreType.DMA((nsteps,)),
            pltpu.SemaphoreType.DMA((nsteps,)),
            pltpu.SemaphoreType.DMA((nsteps,)),
        ],
        compiler_params=pltpu.CompilerParams(collective_id=0),
    )(x, W1, W2)
